# baseline (speedup 1.0000x reference)
import jax
import jax.numpy as jnp
import numpy as np
from jax import lax
from jax.experimental import pallas as pl
from jax.experimental.pallas import tpu as pltpu
from jax.experimental.pallas import tpu_sc as plsc

_N = 36
_PERM = tuple(int(v) for v in np.random.RandomState(0).permutation(_N))

_B = 4096
_D = 128
_NC = 2
_NS = 16
_WIN = 256
_HALF = _N // 2
_NBUF = 3


def _run(x_hbm, out_hbm, bufs, semr, semw, b0, j0):
    def start_in(j, b):
        pltpu.async_copy(
            x_hbm.at[_PERM[j0 + j], pl.ds(b0, _WIN), :], bufs[b], semr[b]
        )

    def wait_in(j, b):
        pltpu.make_async_copy(
            x_hbm.at[_PERM[j0 + j], pl.ds(b0, _WIN), :], bufs[b], semr[b]
        ).wait()

    def start_out(j, b):
        pltpu.async_copy(
            bufs[b], out_hbm.at[j0 + j, pl.ds(b0, _WIN), :], semw[b]
        )

    def wait_out(j, b):
        pltpu.make_async_copy(
            bufs[b], out_hbm.at[j0 + j, pl.ds(b0, _WIN), :], semw[b]
        ).wait()

    for b in range(_NBUF):
        start_in(b, b)

    for t in range(_HALF):
        b = t % _NBUF
        wait_in(t, b)
        start_out(t, b)
        r = t + 1
        if _NBUF <= r < _HALF:
            rb = r % _NBUF
            wait_out(r - _NBUF, rb)
            start_in(r, rb)

    for t in range(_HALF - _NBUF, _HALF):
        wait_out(t, t % _NBUF)


def _run_spmem(x_hbm, out_hbm, sbuf, semr, semw, j0):
    def start_in(j, b):
        pltpu.async_copy(x_hbm.at[_PERM[j0 + j]], sbuf.at[b], semr[b])

    def wait_in(j, b):
        pltpu.make_async_copy(
            x_hbm.at[_PERM[j0 + j]], sbuf.at[b], semr[b]
        ).wait()

    def start_out(j, b):
        pltpu.async_copy(sbuf.at[b], out_hbm.at[j0 + j], semw[b])

    def wait_out(j, b):
        pltpu.make_async_copy(sbuf.at[b], out_hbm.at[j0 + j], semw[b]).wait()

    for b in range(_NBUF):
        start_in(b, b)

    for t in range(_HALF):
        b = t % _NBUF
        wait_in(t, b)
        start_out(t, b)
        r = t + 1
        if _NBUF <= r < _HALF:
            rb = r % _NBUF
            wait_out(r - _NBUF, rb)
            start_in(r, rb)

    for t in range(_HALF - _NBUF, _HALF):
        wait_out(t, t % _NBUF)


def _body(x_hbm, out_hbm, sbuf, semr0, semr1, semr2, semw0, semw1, semw2):
    c = lax.axis_index("c")
    s = lax.axis_index("s")
    semr = (semr0, semr1, semr2)
    semw = (semw0, semw1, semw2)

    @pl.when(s == 0)
    def _():
        @pl.when(c == 0)
        def _():
            _run_spmem(x_hbm, out_hbm, sbuf, semr, semw, 0)

        @pl.when(c == 1)
        def _():
            _run_spmem(x_hbm, out_hbm, sbuf, semr, semw, _HALF)


@jax.jit
def kernel(x):
    xt = jnp.transpose(x, (1, 0, 2))
    mesh = plsc.VectorSubcoreMesh(core_axis_name="c", subcore_axis_name="s")
    out_t = pl.kernel(
        _body,
        out_type=jax.ShapeDtypeStruct((_N, _B, _D), x.dtype),
        mesh=mesh,
        scratch_types=[
            pltpu.VMEM_SHARED((_NBUF, _B, _D), jnp.float32),
            pltpu.SemaphoreType.DMA,
            pltpu.SemaphoreType.DMA,
            pltpu.SemaphoreType.DMA,
            pltpu.SemaphoreType.DMA,
            pltpu.SemaphoreType.DMA,
            pltpu.SemaphoreType.DMA,
        ],
    )(xt)
    return jnp.transpose(out_t, (1, 0, 2))
